# baseline (speedup 1.0000x reference)
import functools

import jax
import jax.numpy as jnp
from jax import lax
from jax.experimental import pallas as pl
from jax.experimental.pallas import tpu as pltpu
from jax.experimental.pallas import tpu_sc as plsc

E, CIN, COUT, D = 8192, 4, 8, 512
L = 16
NC, NS = 2, 16
NW = NC * NS
EPW = E // NW
C = 8
NCH = EPW // C
NB = 2
JPE = D // L
INV_SQRT3 = 0.5773502691896258


def _body(x_hbm, y_hbm, o_hbm, buf, sx0, sx1, sy0, sy1, so0, so1):
    sx = (sx0, sx1)
    sy = (sy0, sy1)
    so = (so0, so1)
    wid = lax.axis_index("s") * NC + lax.axis_index("c")
    base = wid * EPW

    for b in range(NB):
        off = base + b * C
        pltpu.async_copy(x_hbm.at[pl.ds(off, C)], buf.at[b, :, pl.ds(0, CIN)], sx[b])
        pltpu.async_copy(y_hbm.at[pl.ds(off, C)], buf.at[b, :, pl.ds(CIN, CIN)], sy[b])

    def round_body(g, carry):
        for b in range(NB):
            ci = g * NB + b
            off = base + ci * C

            pltpu.make_async_copy(
                x_hbm.at[pl.ds(off, C)], buf.at[b, :, pl.ds(0, CIN)], sx[b]).wait()
            pltpu.make_async_copy(
                y_hbm.at[pl.ds(off, C)], buf.at[b, :, pl.ds(CIN, CIN)], sy[b]).wait()

            def _edge(e, carry3):
                for j in range(JPE):
                    s = pl.ds(j * L, L)
                    x0 = buf[b, e, 0, s]
                    x1 = buf[b, e, 1, s]
                    x2 = buf[b, e, 2, s]
                    x3 = buf[b, e, 3, s]
                    y0 = buf[b, e, 4, s]
                    y1 = buf[b, e, 5, s]
                    y2 = buf[b, e, 6, s]
                    y3 = buf[b, e, 7, s]
                    buf[b, e, 0, s] = x0 * y0
                    buf[b, e, 1, s] = x0 * y1
                    buf[b, e, 2, s] = x0 * y2
                    buf[b, e, 3, s] = x0 * y3
                    buf[b, e, 4, s] = x1 * y0
                    buf[b, e, 5, s] = x2 * y0
                    buf[b, e, 6, s] = x3 * y0
                    buf[b, e, 7, s] = (x1 * y1 + x2 * y2 + x3 * y3) * INV_SQRT3
                return carry3

            lax.fori_loop(0, C, _edge, 0)

            pltpu.async_copy(buf.at[b], o_hbm.at[pl.ds(off, C)], so[b])

            pb = (b - 1) % NB
            pci = ci - 1

            def _drain_refill():
                poff = base + pci * C
                pltpu.make_async_copy(
                    buf.at[pb], o_hbm.at[pl.ds(poff, C)], so[pb]).wait()

                @pl.when(pci + NB < NCH)
                def _():
                    noff = poff + NB * C
                    pltpu.async_copy(
                        x_hbm.at[pl.ds(noff, C)], buf.at[pb, :, pl.ds(0, CIN)], sx[pb])
                    pltpu.async_copy(
                        y_hbm.at[pl.ds(noff, C)], buf.at[pb, :, pl.ds(CIN, CIN)], sy[pb])

            if b == 0:
                pl.when(g > 0)(_drain_refill)
            else:
                _drain_refill()

        return carry

    lax.fori_loop(0, NCH // NB, round_body, 0)

    lb = (NCH - 1) % NB
    loff = base + (NCH - 1) * C
    pltpu.make_async_copy(buf.at[lb], o_hbm.at[pl.ds(loff, C)], so[lb]).wait()


_tp = functools.partial(
    pl.kernel,
    mesh=plsc.VectorSubcoreMesh(core_axis_name="c", subcore_axis_name="s"),
    out_type=jax.ShapeDtypeStruct((E, COUT, D), jnp.float32),
    scratch_types=[
        pltpu.VMEM((NB, C, COUT, D), jnp.float32),
        pltpu.SemaphoreType.DMA,
        pltpu.SemaphoreType.DMA,
        pltpu.SemaphoreType.DMA,
        pltpu.SemaphoreType.DMA,
        pltpu.SemaphoreType.DMA,
        pltpu.SemaphoreType.DMA,
    ],
)(_body)


def kernel(x, y):
    return _tp(x, y)
